# baseline (speedup 1.0000x reference)
import jax
import jax.numpy as jnp
from jax.experimental import pallas as pl
from jax.experimental.pallas import tpu as pltpu

_B = 1024
_K = 100000
_L = 64
_KB = 3072
_NSTEPS = (_K + _KB - 1) // _KB


def _mf_kernel(ut_ref, it_ref, wut_ref, wit_ref, bt_ref, gb_ref, out_ref,
               acc_u, acc_i, acc_b):
    step = pl.program_id(0)

    @pl.when(step == 0)
    def _init():
        acc_u[...] = jnp.zeros_like(acc_u)
        acc_i[...] = jnp.zeros_like(acc_i)
        acc_b[...] = jnp.zeros_like(acc_b)

    def _accumulate(ut, it, wut, wit, bt):
        acc_u[...] += jnp.dot(wut, ut, preferred_element_type=jnp.float32)
        acc_i[...] += jnp.dot(wit, it, preferred_element_type=jnp.float32)
        acc_b[...] += jnp.dot(bt, it, preferred_element_type=jnp.float32)

    @pl.when(step < _NSTEPS - 1)
    def _clean():
        _accumulate(ut_ref[...].astype(jnp.bfloat16),
                    it_ref[...].astype(jnp.bfloat16),
                    wut_ref[...].astype(jnp.bfloat16),
                    wit_ref[...].astype(jnp.bfloat16),
                    bt_ref[...].astype(jnp.bfloat16))

    @pl.when(step == _NSTEPS - 1)
    def _ragged():
        row = jax.lax.broadcasted_iota(jnp.int32, (_KB, 1), 0)
        valid_r = (step * _KB + row) < _K
        col = jax.lax.broadcasted_iota(jnp.int32, (1, _KB), 1)
        valid_c = (step * _KB + col) < _K
        z16 = jnp.bfloat16(0)
        _accumulate(jnp.where(valid_r, ut_ref[...].astype(jnp.bfloat16), z16),
                    jnp.where(valid_r, it_ref[...].astype(jnp.bfloat16), z16),
                    jnp.where(valid_c, wut_ref[...].astype(jnp.bfloat16), z16),
                    jnp.where(valid_c, wit_ref[...].astype(jnp.bfloat16), z16),
                    jnp.where(valid_c, bt_ref[...].astype(jnp.bfloat16), z16))

    @pl.when(step == _NSTEPS - 1)
    def _finalize():
        inter = jnp.sum(acc_u[...] * acc_i[...], axis=0, keepdims=True)
        out_ref[...] = inter + acc_b[...] + gb_ref[0]


def kernel(user_features, item_features, user_latent_weight,
           item_latent_weight, item_biases_weight, global_bias):
    ut = user_features.T
    it = item_features.T
    wut = user_latent_weight.T
    wit = item_latent_weight.T
    bt = item_biases_weight.reshape(1, _K)
    out = pl.pallas_call(
        _mf_kernel,
        grid=(_NSTEPS,),
        in_specs=[
            pl.BlockSpec((_KB, _B), lambda k: (k, 0)),
            pl.BlockSpec((_KB, _B), lambda k: (k, 0)),
            pl.BlockSpec((_L, _KB), lambda k: (0, k)),
            pl.BlockSpec((_L, _KB), lambda k: (0, k)),
            pl.BlockSpec((1, _KB), lambda k: (0, k)),
            pl.BlockSpec(memory_space=pltpu.SMEM),
        ],
        out_specs=pl.BlockSpec((1, _B), lambda k: (0, 0)),
        out_shape=jax.ShapeDtypeStruct((1, _B), jnp.float32),
        scratch_shapes=[
            pltpu.VMEM((_L, _B), jnp.float32),
            pltpu.VMEM((_L, _B), jnp.float32),
            pltpu.VMEM((1, _B), jnp.float32),
        ],
        compiler_params=pltpu.CompilerParams(
            dimension_semantics=("arbitrary",),
        ),
    )(ut, it, wut, wit, bt, global_bias)
    return out.reshape(_B)


# speedup vs baseline: 1.0416x; 1.0416x over previous
import jax
import jax.numpy as jnp
from jax.experimental import pallas as pl
from jax.experimental.pallas import tpu as pltpu

_B = 1024
_K = 100000
_L = 64
_KB = 1024
_NSTEPS = (_K + _KB - 1) // _KB


def _mf_kernel(ut_ref, it_ref, wut_ref, wit_ref, bt_ref, gb_ref, out_ref,
               acc_u, acc_i, acc_b):
    step = pl.program_id(0)

    @pl.when(step == 0)
    def _init():
        acc_u[...] = jnp.zeros_like(acc_u)
        acc_i[...] = jnp.zeros_like(acc_i)
        acc_b[...] = jnp.zeros_like(acc_b)

    def _accumulate(ut, it, wut, wit, bt):
        acc_u[...] += jnp.dot(wut, ut, preferred_element_type=jnp.float32)
        acc_i[...] += jnp.dot(wit, it, preferred_element_type=jnp.float32)
        acc_b[...] += jnp.dot(bt, it, preferred_element_type=jnp.float32)

    @pl.when(step < _NSTEPS - 1)
    def _clean():
        _accumulate(ut_ref[...].astype(jnp.bfloat16),
                    it_ref[...].astype(jnp.bfloat16),
                    wut_ref[...].astype(jnp.bfloat16),
                    wit_ref[...].astype(jnp.bfloat16),
                    bt_ref[...].astype(jnp.bfloat16))

    @pl.when(step == _NSTEPS - 1)
    def _ragged():
        row = jax.lax.broadcasted_iota(jnp.int32, (_KB, 1), 0)
        valid_r = (step * _KB + row) < _K
        col = jax.lax.broadcasted_iota(jnp.int32, (1, _KB), 1)
        valid_c = (step * _KB + col) < _K
        z16 = jnp.bfloat16(0)
        _accumulate(jnp.where(valid_r, ut_ref[...].astype(jnp.bfloat16), z16),
                    jnp.where(valid_r, it_ref[...].astype(jnp.bfloat16), z16),
                    jnp.where(valid_c, wut_ref[...].astype(jnp.bfloat16), z16),
                    jnp.where(valid_c, wit_ref[...].astype(jnp.bfloat16), z16),
                    jnp.where(valid_c, bt_ref[...].astype(jnp.bfloat16), z16))

    @pl.when(step == _NSTEPS - 1)
    def _finalize():
        inter = jnp.sum(acc_u[...] * acc_i[...], axis=0, keepdims=True)
        out_ref[...] = inter + acc_b[...] + gb_ref[0]


def kernel(user_features, item_features, user_latent_weight,
           item_latent_weight, item_biases_weight, global_bias):
    ut = user_features.T
    it = item_features.T
    wut = user_latent_weight.T
    wit = item_latent_weight.T
    bt = item_biases_weight.reshape(1, _K)
    out = pl.pallas_call(
        _mf_kernel,
        grid=(_NSTEPS,),
        in_specs=[
            pl.BlockSpec((_KB, _B), lambda k: (k, 0)),
            pl.BlockSpec((_KB, _B), lambda k: (k, 0)),
            pl.BlockSpec((_L, _KB), lambda k: (0, k)),
            pl.BlockSpec((_L, _KB), lambda k: (0, k)),
            pl.BlockSpec((1, _KB), lambda k: (0, k)),
            pl.BlockSpec(memory_space=pltpu.SMEM),
        ],
        out_specs=pl.BlockSpec((1, _B), lambda k: (0, 0)),
        out_shape=jax.ShapeDtypeStruct((1, _B), jnp.float32),
        scratch_shapes=[
            pltpu.VMEM((_L, _B), jnp.float32),
            pltpu.VMEM((_L, _B), jnp.float32),
            pltpu.VMEM((1, _B), jnp.float32),
        ],
        compiler_params=pltpu.CompilerParams(
            dimension_semantics=("arbitrary",),
        ),
    )(ut, it, wut, wit, bt, global_bias)
    return out.reshape(_B)
